# baseline (speedup 1.0000x reference)
import functools

import jax
import jax.numpy as jnp
from jax import lax
from jax.experimental import pallas as pl
from jax.experimental.pallas import tpu as pltpu
from jax.experimental.pallas import tpu_sc as plsc

CS = 72
ROWW = CS * 3
NPIX = CS * ROWW
L = 16
NC = 2
NS = 16
NW = NC * NS
NBUF = 4
CROW_PAD = 224


def _bounds(posv, i):
    row = posv[i]
    return row[0], row[1], row[2], row[3]


def _paint(posv, colrow, cv, i, lane, v255, colored):
    r_lo, r_hi, c_lo3, c_hi3 = _bounds(posv, i)

    @pl.when((r_hi > r_lo) & (c_hi3 > c_lo3))
    def _():
        k0 = lax.div(c_lo3, L)
        k1 = lax.div(c_hi3 + (L - 1), L)

        def row_body(r, carry):
            rb = r * ROWW

            def chunk_body(k, c2):
                w = k * L + lane
                m = (w >= c_lo3) & (w < c_hi3)
                if colored:
                    v = colrow[pl.ds(k * L, L)]
                else:
                    v = v255
                plsc.store_scatter(cv, [rb + w], v, mask=m)
                return c2

            lax.fori_loop(k0, k1, chunk_body, 0)
            return carry

        lax.fori_loop(r_lo, r_hi, row_body, 0)


def _build_colrow(posv, colv, colrow, i, lane):
    r_lo, r_hi, c_lo3, c_hi3 = _bounds(posv, i)

    @pl.when((r_hi > r_lo) & (c_hi3 > c_lo3))
    def _():
        cbase = i * 3
        pats = [
            plsc.load_gather(
                colv, [cbase + (jnp.full((L,), 16 * j, jnp.int32) + lane) % 3]
            )
            for j in range(3)
        ]
        for j in range(CROW_PAD // L):
            colrow[pl.ds(j * L, L)] = pats[j % 3]


def _sc_body(pos_hbm, col_hbm, out_hbm, posv, colv, colrow,
             cv0, cv1, cv2, cv3, s0, s1, s2, s3):
    spw = out_hbm.shape[0] // NW
    wid = lax.axis_index("s") * NC + lax.axis_index("c")
    base = wid * spw

    pltpu.sync_copy(pos_hbm.at[pl.ds(base, spw)], posv)
    pltpu.sync_copy(col_hbm.at[pl.ds(base * 3, spw * 3)], colv)

    cvs = [cv0, cv1, cv2, cv3]
    sems = [s0, s1, s2, s3]
    lane = lax.iota(jnp.int32, L)
    v255 = jnp.full((L,), 255.0, jnp.float32)

    def fill(t, carry):
        for cv in cvs:
            for u in range(4):
                cv[pl.ds((t * 4 + u) * L, L)] = v255
        return carry

    lax.fori_loop(0, NPIX // (4 * L), fill, 0)

    for b in range(NBUF):
        _build_colrow(posv, colv, colrow, b, lane)
        _paint(posv, colrow, cvs[b], b, lane, v255, colored=True)
        pltpu.async_copy(cvs[b], out_hbm.at[base + b], sems[b])

    def group(g, carry):
        for b in range(NBUF):
            i = g * NBUF + b
            prev = i - NBUF
            pltpu.make_async_copy(cvs[b], out_hbm.at[base + prev], sems[b]).wait()
            _paint(posv, colrow, cvs[b], prev, lane, v255, colored=False)
            _build_colrow(posv, colv, colrow, i, lane)
            _paint(posv, colrow, cvs[b], i, lane, v255, colored=True)
            pltpu.async_copy(cvs[b], out_hbm.at[base + i], sems[b])
        return carry

    lax.fori_loop(1, spw // NBUF, group, 0)

    for b in range(NBUF):
        last = spw - NBUF + b
        pltpu.make_async_copy(cvs[b], out_hbm.at[base + last], sems[b]).wait()


@jax.jit
def kernel(positions, colors):
    pos = positions.astype(jnp.int32)
    b = pos.shape[0]
    r_lo = jnp.minimum(pos[:, 0, 0], CS)
    r_hi = jnp.minimum(pos[:, 1, 0], CS)
    c_lo = jnp.minimum(pos[:, 0, 1], CS)
    c_hi = jnp.minimum(pos[:, 1, 1], CS)
    pos4 = jnp.stack([r_lo, r_hi, c_lo * 3, c_hi * 3], axis=1)
    pos16 = jnp.zeros((b, L), jnp.int32).at[:, :4].set(pos4)

    spw = b // NW
    assert spw * NW == b and spw % NBUF == 0

    sc_call = pl.kernel(
        _sc_body,
        out_type=jax.ShapeDtypeStruct((b, NPIX), jnp.float32),
        mesh=plsc.VectorSubcoreMesh(core_axis_name="c", subcore_axis_name="s"),
        compiler_params=pltpu.CompilerParams(needs_layout_passes=False),
        scratch_types=[
            pltpu.VMEM((spw, L), jnp.int32),
            pltpu.VMEM((spw * 3,), jnp.float32),
            pltpu.VMEM((CROW_PAD,), jnp.float32),
            pltpu.VMEM((NPIX,), jnp.float32),
            pltpu.VMEM((NPIX,), jnp.float32),
            pltpu.VMEM((NPIX,), jnp.float32),
            pltpu.VMEM((NPIX,), jnp.float32),
            pltpu.SemaphoreType.DMA,
            pltpu.SemaphoreType.DMA,
            pltpu.SemaphoreType.DMA,
            pltpu.SemaphoreType.DMA,
        ],
    )
    out = sc_call(pos16, colors.reshape(-1))
    return out.reshape(b, CS, CS, 3)


# speedup vs baseline: 2.9252x; 2.9252x over previous
import functools

import jax
import jax.numpy as jnp
from jax import lax
from jax.experimental import pallas as pl
from jax.experimental.pallas import tpu as pltpu
from jax.experimental.pallas import tpu_sc as plsc

CS = 72
CT = 9
L = 16
NC = 2
NS = 16
NW = NC * NS
SPW = 128
NBUF = 3
PLANE = (27, 8, 128)


def _paint_sample(posv, colv, buf, lb, r, lane, v255, colored):
    prow = posv[lb]
    c_lo = prow[2]
    c_hi = prow[3]
    lbv = jnp.full((L,), 0, jnp.int32) + lb
    if colored:
        crow = colv[lb]
        vals = [jnp.full((L,), 0.0, jnp.float32) + crow[ch] for ch in range(3)]
    else:
        vals = [v255, v255, v255]
    nk = lax.shift_right_logical(c_hi - c_lo + (L - 1), 4)

    def ck(k2, carry):
        c = c_lo + k2 * L + lane
        msk = c < c_hi
        ct = lax.shift_right_logical(c, 3)
        c8 = c & 7
        for ch in range(3):
            plsc.store_scatter(buf, [ct + 9 * ch, c8, lbv], vals[ch], mask=msk)
        return carry

    lax.fori_loop(0, nk, ck, 0)


def _do_row(posv, colv, rlov, rhiv, buf, r, lane, v255, colored):
    for k in range(SPW // L):
        rlo = rlov[pl.ds(k * L, L)]
        rhi = rhiv[pl.ds(k * L, L)]
        m = (r >= rlo) & (r < rhi)
        cnt = plsc.all_reduce_population_count(m)[0]

        def body(t, mc):
            la = plsc.all_reduce_ffs(mc)[0]
            _paint_sample(posv, colv, buf, k * L + la, r, lane, v255, colored)
            return mc & (lane != la)

        lax.fori_loop(0, cnt, body, m)


def _sc_body(pos_hbm, soa_hbm, col_hbm, bg_hbm, out6, posv, colv,
             rlov, rhiv, buf0, buf1, buf2, s0, s1, s2):
    o = out6.reshape(CS * 27, NW, 8, 128)
    wid = lax.axis_index("s") * NC + lax.axis_index("c")
    base = wid * SPW
    nb = 4096

    pltpu.sync_copy(pos_hbm.at[pl.ds(base, SPW)], posv)
    pltpu.sync_copy(col_hbm.at[pl.ds(base, SPW)], colv)
    pltpu.sync_copy(soa_hbm.at[pl.ds(base, SPW)], rlov)
    pltpu.sync_copy(soa_hbm.at[pl.ds(nb + base, SPW)], rhiv)

    bufs = [buf0, buf1, buf2]
    sems = [s0, s1, s2]
    lane = lax.iota(jnp.int32, L)
    v255 = jnp.full((L,), 255.0, jnp.float32)

    for buf in bufs:
        pltpu.sync_copy(bg_hbm, buf)

    for b in range(NBUF):
        _do_row(posv, colv, rlov, rhiv, bufs[b], b, lane, v255, colored=True)
        pltpu.async_copy(bufs[b], o.at[pl.ds(b * 27, 27), wid], sems[b])

    def group(g, carry):
        for b in range(NBUF):
            r = g * NBUF + b
            prev = r - NBUF
            pltpu.make_async_copy(
                bufs[b], o.at[pl.ds(prev * 27, 27), wid], sems[b]
            ).wait()
            _do_row(posv, colv, rlov, rhiv, bufs[b], prev, lane, v255,
                    colored=False)
            _do_row(posv, colv, rlov, rhiv, bufs[b], r, lane, v255,
                    colored=True)
            pltpu.async_copy(bufs[b], o.at[pl.ds(r * 27, 27), wid], sems[b])
        return carry

    lax.fori_loop(1, CS // NBUF, group, 0)

    for b in range(NBUF):
        last = CS - NBUF + b
        pltpu.make_async_copy(
            bufs[b], o.at[pl.ds(last * 27, 27), wid], sems[b]
        ).wait()


@jax.jit
def kernel(positions, colors):
    pos = positions.astype(jnp.int32)
    nb = pos.shape[0]
    r_lo = jnp.minimum(pos[:, 0, 0], CS)
    r_hi = jnp.minimum(pos[:, 1, 0], CS)
    c_lo = jnp.minimum(pos[:, 0, 1], CS)
    c_hi = jnp.minimum(pos[:, 1, 1], CS)
    r_hi_eff = jnp.where(c_hi > c_lo, r_hi, 0)

    pos4 = jnp.stack([r_lo, r_hi_eff, c_lo, c_hi], axis=1)
    pos16 = jnp.zeros((nb, L), jnp.int32).at[:, :4].set(pos4)
    soa = jnp.concatenate([r_lo, r_hi_eff])
    col16 = jnp.zeros((nb, L), jnp.float32).at[:, :3].set(colors)
    bg = jnp.full(PLANE, 255.0, jnp.float32)

    assert nb == SPW * NW

    sc_call = pl.kernel(
        _sc_body,
        out_type=jax.ShapeDtypeStruct((CS, 3, CT, NW, 8, 128), jnp.float32),
        mesh=plsc.VectorSubcoreMesh(core_axis_name="c", subcore_axis_name="s"),
        compiler_params=pltpu.CompilerParams(needs_layout_passes=False),
        scratch_types=[
            pltpu.VMEM((SPW, L), jnp.int32),
            pltpu.VMEM((SPW, L), jnp.float32),
            pltpu.VMEM((SPW,), jnp.int32),
            pltpu.VMEM((SPW,), jnp.int32),
            pltpu.VMEM(PLANE, jnp.float32),
            pltpu.VMEM(PLANE, jnp.float32),
            pltpu.VMEM(PLANE, jnp.float32),
            pltpu.SemaphoreType.DMA,
            pltpu.SemaphoreType.DMA,
            pltpu.SemaphoreType.DMA,
        ],
    )
    out6 = sc_call(pos16, soa, col16.reshape(nb, L), bg)
    return jnp.transpose(out6, (3, 5, 0, 2, 4, 1)).reshape(nb, CS, CS, 3)


# speedup vs baseline: 6.2321x; 2.1305x over previous
import functools

import jax
import jax.numpy as jnp
from jax import lax
from jax.experimental import pallas as pl
from jax.experimental.pallas import tpu as pltpu
from jax.experimental.pallas import tpu_sc as plsc

CS = 72
CT = 9
L = 16
NC = 2
NS = 16
NW = NC * NS
SPW = 128
NBUF = 3
PLANE = (27, 8, 128)


def _paint_sample(posv, colv, buf, lb, lane, v255, colored):
    prow = posv[lb]
    c_lo = prow[2]
    c_hi = prow[3]
    lbv = jnp.full((L,), 0, jnp.int32) + lb
    if colored:
        crow = colv[lb]
        vals = [jnp.full((L,), 0.0, jnp.float32) + crow[ch] for ch in range(3)]
    else:
        vals = [v255, v255, v255]
    nk = lax.shift_right_logical(c_hi - c_lo + (L - 1), 4)

    def ck(k2, carry):
        c = c_lo + k2 * L + lane
        msk = c < c_hi
        ct = lax.shift_right_logical(c, 3)
        c8 = c & 7
        for ch in range(3):
            plsc.store_scatter(buf, [ct + 9 * ch, c8, lbv], vals[ch], mask=msk)
        return carry

    lax.fori_loop(0, nk, ck, 0)


def _enumerate(posv, colv, buf, m, k, lane, v255, colored):
    cnt = plsc.all_reduce_population_count(m)[0]

    def body(t, mc):
        la = plsc.all_reduce_ffs(mc)[0]
        _paint_sample(posv, colv, buf, k * L + la, lane, v255, colored)
        return mc & (lane != la)

    lax.fori_loop(0, cnt, body, m)


def _update_row(posv, colv, rlov, rhiv, buf, r, prevr, lane, v255):
    for k in range(SPW // L):
        rlo = rlov[pl.ds(k * L, L)]
        rhi = rhiv[pl.ds(k * L, L)]
        act_r = (r >= rlo) & (r < rhi)
        if prevr is None:
            born = act_r
        else:
            born = act_r & (rlo > prevr)
            dead = (prevr >= rlo) & (prevr < rhi) & (rhi <= r)
            _enumerate(posv, colv, buf, dead, k, lane, v255, colored=False)
        _enumerate(posv, colv, buf, born, k, lane, v255, colored=True)


def _sc_body(pos_hbm, soa_hbm, col_hbm, bg_hbm, out6, posv, colv,
             rlov, rhiv, buf0, buf1, buf2, s0, s1, s2):
    o = out6.reshape(CS * 27, NW, 8, 128)
    wid = lax.axis_index("s") * NC + lax.axis_index("c")
    base = wid * SPW
    nb = 4096

    pltpu.sync_copy(pos_hbm.at[pl.ds(base, SPW)], posv)
    pltpu.sync_copy(col_hbm.at[pl.ds(base, SPW)], colv)
    pltpu.sync_copy(soa_hbm.at[pl.ds(base, SPW)], rlov)
    pltpu.sync_copy(soa_hbm.at[pl.ds(nb + base, SPW)], rhiv)

    bufs = [buf0, buf1, buf2]
    sems = [s0, s1, s2]
    lane = lax.iota(jnp.int32, L)
    v255 = jnp.full((L,), 255.0, jnp.float32)

    for buf in bufs:
        pltpu.sync_copy(bg_hbm, buf)

    for b in range(NBUF):
        _update_row(posv, colv, rlov, rhiv, bufs[b], b, None, lane, v255)
        pltpu.async_copy(bufs[b], o.at[pl.ds(b * 27, 27), wid], sems[b])

    def group(g, carry):
        for b in range(NBUF):
            r = g * NBUF + b
            prev = r - NBUF
            pltpu.make_async_copy(
                bufs[b], o.at[pl.ds(prev * 27, 27), wid], sems[b]
            ).wait()
            _update_row(posv, colv, rlov, rhiv, bufs[b], r, prev, lane, v255)
            pltpu.async_copy(bufs[b], o.at[pl.ds(r * 27, 27), wid], sems[b])
        return carry

    lax.fori_loop(1, CS // NBUF, group, 0)

    for b in range(NBUF):
        last = CS - NBUF + b
        pltpu.make_async_copy(
            bufs[b], o.at[pl.ds(last * 27, 27), wid], sems[b]
        ).wait()


@jax.jit
def kernel(positions, colors):
    pos = positions.astype(jnp.int32)
    nb = pos.shape[0]
    r_lo = jnp.minimum(pos[:, 0, 0], CS)
    r_hi = jnp.minimum(pos[:, 1, 0], CS)
    c_lo = jnp.minimum(pos[:, 0, 1], CS)
    c_hi = jnp.minimum(pos[:, 1, 1], CS)
    r_hi_eff = jnp.where(c_hi > c_lo, r_hi, 0)

    pos4 = jnp.stack([r_lo, r_hi_eff, c_lo, c_hi], axis=1)
    pos16 = jnp.zeros((nb, L), jnp.int32).at[:, :4].set(pos4)
    soa = jnp.concatenate([r_lo, r_hi_eff])
    col16 = jnp.zeros((nb, L), jnp.float32).at[:, :3].set(colors)
    bg = jnp.full(PLANE, 255.0, jnp.float32)

    assert nb == SPW * NW

    sc_call = pl.kernel(
        _sc_body,
        out_type=jax.ShapeDtypeStruct((CS, 3, CT, NW, 8, 128), jnp.float32),
        mesh=plsc.VectorSubcoreMesh(core_axis_name="c", subcore_axis_name="s"),
        compiler_params=pltpu.CompilerParams(needs_layout_passes=False),
        scratch_types=[
            pltpu.VMEM((SPW, L), jnp.int32),
            pltpu.VMEM((SPW, L), jnp.float32),
            pltpu.VMEM((SPW,), jnp.int32),
            pltpu.VMEM((SPW,), jnp.int32),
            pltpu.VMEM(PLANE, jnp.float32),
            pltpu.VMEM(PLANE, jnp.float32),
            pltpu.VMEM(PLANE, jnp.float32),
            pltpu.SemaphoreType.DMA,
            pltpu.SemaphoreType.DMA,
            pltpu.SemaphoreType.DMA,
        ],
    )
    out6 = sc_call(pos16, soa, col16.reshape(nb, L), bg)
    return jnp.transpose(out6, (3, 5, 0, 2, 4, 1)).reshape(nb, CS, CS, 3)


# speedup vs baseline: 6.8820x; 1.1043x over previous
import functools

import jax
import jax.numpy as jnp
from jax import lax
from jax.experimental import pallas as pl
from jax.experimental.pallas import tpu as pltpu
from jax.experimental.pallas import tpu_sc as plsc

CS = 72
CT = 9
L = 16
NC = 2
NS = 16
NW = NC * NS
SPW = 128
NBUF = 3
PLANE = (27, 8, 128)


def _paint_sample(posv, colv, buf, lb, r, lane, v255):
    prow = posv[lb]
    c_lo = prow[2]
    c_hi = prow[3]
    born = r < prow[1]
    lbv = jnp.full((L,), 0, jnp.int32) + lb
    crow = colv[lb]
    vals = [
        jnp.where(born, jnp.full((L,), 0.0, jnp.float32) + crow[ch], v255)
        for ch in range(3)
    ]
    nk = lax.shift_right_logical(c_hi - c_lo + (L - 1), 4)

    def ck(k2, carry):
        c = c_lo + k2 * L + lane
        msk = c < c_hi
        ct = lax.shift_right_logical(c, 3)
        c8 = c & 7
        for ch in range(3):
            plsc.store_scatter(buf, [ct + 9 * ch, c8, lbv], vals[ch], mask=msk)
        return carry

    lax.fori_loop(0, nk, ck, 0)


def _update_row(posv, colv, rlov, rhiv, buf, r, prevr, lane, v255):
    for k in range(SPW // L):
        rlo = rlov[pl.ds(k * L, L)]
        rhi = rhiv[pl.ds(k * L, L)]
        act_r = (r >= rlo) & (r < rhi)
        if prevr is None:
            ev = act_r
        else:
            born = act_r & (rlo > prevr)
            dead = (prevr >= rlo) & (prevr < rhi) & (rhi <= r)
            ev = born | dead
        cnt = plsc.all_reduce_population_count(ev)[0]

        def body(t, mc):
            la = plsc.all_reduce_ffs(mc)[0]
            _paint_sample(posv, colv, buf, k * L + la, r, lane, v255)
            return mc & (lane != la)

        lax.fori_loop(0, cnt, body, ev)


def _sc_body(pos_hbm, soa_hbm, col_hbm, bg_hbm, out6, posv, colv,
             rlov, rhiv, buf0, buf1, buf2, s0, s1, s2):
    o = out6.reshape(CS * 27, NW, 8, 128)
    wid = lax.axis_index("s") * NC + lax.axis_index("c")
    base = wid * SPW
    nb = 4096

    pltpu.sync_copy(pos_hbm.at[pl.ds(base, SPW)], posv)
    pltpu.sync_copy(col_hbm.at[pl.ds(base, SPW)], colv)
    pltpu.sync_copy(soa_hbm.at[pl.ds(base, SPW)], rlov)
    pltpu.sync_copy(soa_hbm.at[pl.ds(nb + base, SPW)], rhiv)

    bufs = [buf0, buf1, buf2]
    sems = [s0, s1, s2]
    lane = lax.iota(jnp.int32, L)
    v255 = jnp.full((L,), 255.0, jnp.float32)

    for buf in bufs:
        pltpu.sync_copy(bg_hbm, buf)

    for b in range(NBUF):
        _update_row(posv, colv, rlov, rhiv, bufs[b], b, None, lane, v255)
        pltpu.async_copy(bufs[b], o.at[pl.ds(b * 27, 27), wid], sems[b])

    def group(g, carry):
        for b in range(NBUF):
            r = g * NBUF + b
            prev = r - NBUF
            pltpu.make_async_copy(
                bufs[b], o.at[pl.ds(prev * 27, 27), wid], sems[b]
            ).wait()
            _update_row(posv, colv, rlov, rhiv, bufs[b], r, prev, lane, v255)
            pltpu.async_copy(bufs[b], o.at[pl.ds(r * 27, 27), wid], sems[b])
        return carry

    lax.fori_loop(1, CS // NBUF, group, 0)

    for b in range(NBUF):
        last = CS - NBUF + b
        pltpu.make_async_copy(
            bufs[b], o.at[pl.ds(last * 27, 27), wid], sems[b]
        ).wait()


@jax.jit
def kernel(positions, colors):
    pos = positions.astype(jnp.int32)
    nb = pos.shape[0]
    r_lo = jnp.minimum(pos[:, 0, 0], CS)
    r_hi = jnp.minimum(pos[:, 1, 0], CS)
    c_lo = jnp.minimum(pos[:, 0, 1], CS)
    c_hi = jnp.minimum(pos[:, 1, 1], CS)
    r_hi_eff = jnp.where(c_hi > c_lo, r_hi, 0)

    pos4 = jnp.stack([r_lo, r_hi_eff, c_lo, c_hi], axis=1)
    pos16 = jnp.zeros((nb, L), jnp.int32).at[:, :4].set(pos4)
    soa = jnp.concatenate([r_lo, r_hi_eff])
    col16 = jnp.zeros((nb, L), jnp.float32).at[:, :3].set(colors)
    bg = jnp.full(PLANE, 255.0, jnp.float32)

    assert nb == SPW * NW

    sc_call = pl.kernel(
        _sc_body,
        out_type=jax.ShapeDtypeStruct((CS, 3, CT, NW, 8, 128), jnp.float32),
        mesh=plsc.VectorSubcoreMesh(core_axis_name="c", subcore_axis_name="s"),
        compiler_params=pltpu.CompilerParams(needs_layout_passes=False),
        scratch_types=[
            pltpu.VMEM((SPW, L), jnp.int32),
            pltpu.VMEM((SPW, L), jnp.float32),
            pltpu.VMEM((SPW,), jnp.int32),
            pltpu.VMEM((SPW,), jnp.int32),
            pltpu.VMEM(PLANE, jnp.float32),
            pltpu.VMEM(PLANE, jnp.float32),
            pltpu.VMEM(PLANE, jnp.float32),
            pltpu.SemaphoreType.DMA,
            pltpu.SemaphoreType.DMA,
            pltpu.SemaphoreType.DMA,
        ],
    )
    out6 = sc_call(pos16, soa, col16.reshape(nb, L), bg)
    return jnp.transpose(out6, (3, 5, 0, 2, 4, 1)).reshape(nb, CS, CS, 3)


# speedup vs baseline: 7.3591x; 1.0693x over previous
import functools

import jax
import jax.numpy as jnp
from jax import lax
from jax.experimental import pallas as pl
from jax.experimental.pallas import tpu as pltpu
from jax.experimental.pallas import tpu_sc as plsc

CS = 72
CT = 9
L = 16
NC = 2
NS = 16
NW = NC * NS
SPW = 128
NBUF = 3
PLANE = (27, 8, 128)


def _paint_sample(posv, colv, buf, lb, r, lane, v255):
    prow = posv[lb]
    c_lo = prow[2]
    c_hi = prow[3]
    born = r < prow[1]
    lbv = jnp.full((L,), 0, jnp.int32) + lb
    crow = colv[lb]
    vals = [
        jnp.where(born, jnp.full((L,), 0.0, jnp.float32) + crow[ch], v255)
        for ch in range(3)
    ]
    nk = lax.shift_right_logical(c_hi - c_lo + (L - 1), 4)

    def ck(k2, carry):
        c = c_lo + k2 * L + lane
        msk = c < c_hi
        ct = lax.shift_right_logical(c, 3)
        c8 = c & 7
        for ch in range(3):
            plsc.store_scatter(buf, [ct + 9 * ch, c8, lbv], vals[ch], mask=msk)
        return carry

    lax.fori_loop(0, nk, ck, 0)


def _update_row(posv, colv, rlov, rhiv, buf, r, prevr, lane, v255):
    for k in range(SPW // L):
        rlo = rlov[pl.ds(k * L, L)]
        rhi = rhiv[pl.ds(k * L, L)]
        act_r = (r >= rlo) & (r < rhi)
        if prevr is None:
            ev = act_r
        else:
            born = act_r & (rlo > prevr)
            dead = (prevr >= rlo) & (prevr < rhi) & (rhi <= r)
            ev = born | dead
        cnt = plsc.all_reduce_population_count(ev)[0]

        def body(t, mc):
            la = plsc.all_reduce_ffs(mc)[0]
            _paint_sample(posv, colv, buf, k * L + la, r, lane, v255)
            return mc & (lane != la)

        lax.fori_loop(0, cnt, body, ev)


def _sc_body(pos_hbm, soa_hbm, col_hbm, bg_hbm, out6, posv, colv,
             rlov, rhiv, buf0, buf1, buf2, s0, s1, s2):
    o = out6.reshape(CS * 27, NW, 8, 128)
    wid = lax.axis_index("s") * NC + lax.axis_index("c")
    base = wid * SPW
    nb = 4096

    bufs = [buf0, buf1, buf2]
    sems = [s0, s1, s2]
    lane = lax.iota(jnp.int32, L)
    v255 = jnp.full((L,), 255.0, jnp.float32)

    stages = [
        (pos_hbm.at[pl.ds(base, SPW)], posv),
        (col_hbm.at[pl.ds(base, SPW)], colv),
        (soa_hbm.at[pl.ds(base, SPW)], rlov),
        (soa_hbm.at[pl.ds(nb + base, SPW)], rhiv),
        (bg_hbm, buf0),
        (bg_hbm, buf1),
        (bg_hbm, buf2),
    ]
    for i, (src, dst) in enumerate(stages):
        pltpu.async_copy(src, dst, sems[i % NBUF])
    for i, (src, dst) in enumerate(stages):
        pltpu.make_async_copy(src, dst, sems[i % NBUF]).wait()

    for b in range(NBUF):
        _update_row(posv, colv, rlov, rhiv, bufs[b], b, None, lane, v255)
        pltpu.async_copy(bufs[b], o.at[pl.ds(b * 27, 27), wid], sems[b])

    def group(g, carry):
        for b in range(NBUF):
            r = g * NBUF + b
            prev = r - NBUF
            pltpu.make_async_copy(
                bufs[b], o.at[pl.ds(prev * 27, 27), wid], sems[b]
            ).wait()
            _update_row(posv, colv, rlov, rhiv, bufs[b], r, prev, lane, v255)
            pltpu.async_copy(bufs[b], o.at[pl.ds(r * 27, 27), wid], sems[b])
        return carry

    lax.fori_loop(1, CS // NBUF, group, 0)

    for b in range(NBUF):
        last = CS - NBUF + b
        pltpu.make_async_copy(
            bufs[b], o.at[pl.ds(last * 27, 27), wid], sems[b]
        ).wait()


@jax.jit
def kernel(positions, colors):
    pos = positions.astype(jnp.int32)
    nb = pos.shape[0]
    r_lo = jnp.minimum(pos[:, 0, 0], CS)
    r_hi = jnp.minimum(pos[:, 1, 0], CS)
    c_lo = jnp.minimum(pos[:, 0, 1], CS)
    c_hi = jnp.minimum(pos[:, 1, 1], CS)
    r_hi_eff = jnp.where(c_hi > c_lo, r_hi, 0)

    pos4 = jnp.stack([r_lo, r_hi_eff, c_lo, c_hi], axis=1)
    pos16 = jnp.zeros((nb, L), jnp.int32).at[:, :4].set(pos4)
    soa = jnp.concatenate([r_lo, r_hi_eff])
    col16 = jnp.zeros((nb, L), jnp.float32).at[:, :3].set(colors)
    bg = jnp.full(PLANE, 255.0, jnp.float32)

    assert nb == SPW * NW

    sc_call = pl.kernel(
        _sc_body,
        out_type=jax.ShapeDtypeStruct((CS, 3, CT, NW, 8, 128), jnp.float32),
        mesh=plsc.VectorSubcoreMesh(core_axis_name="c", subcore_axis_name="s"),
        compiler_params=pltpu.CompilerParams(needs_layout_passes=False),
        scratch_types=[
            pltpu.VMEM((SPW, L), jnp.int32),
            pltpu.VMEM((SPW, L), jnp.float32),
            pltpu.VMEM((SPW,), jnp.int32),
            pltpu.VMEM((SPW,), jnp.int32),
            pltpu.VMEM(PLANE, jnp.float32),
            pltpu.VMEM(PLANE, jnp.float32),
            pltpu.VMEM(PLANE, jnp.float32),
            pltpu.SemaphoreType.DMA,
            pltpu.SemaphoreType.DMA,
            pltpu.SemaphoreType.DMA,
        ],
    )
    out6 = sc_call(pos16, soa, col16.reshape(nb, L), bg)
    return jnp.transpose(out6, (3, 5, 0, 2, 4, 1)).reshape(nb, CS, CS, 3)
